# baseline (speedup 1.0000x reference)
import jax
import jax.numpy as jnp
from jax import lax
from jax.experimental import pallas as pl
from jax.experimental.pallas import tpu as pltpu
from jax.experimental.pallas import tpu_sc as plsc

ENT_TOTAL = 100000
REL_TOTAL = 1000
HIDDEN = 64
HALF = HIDDEN // 2
BATCH = 16384
LMBDA = 0.0001

NC = 1
NS = 16
LANES = 16
NW = NC * NS
ROWS_PER_W = BATCH // NW
CHUNK = 128
NCHUNK = ROWS_PER_W // CHUNK
NBUF = 2


def _mxu_t(x, k):
    eye = jnp.eye(k, dtype=jnp.float32)
    return jax.lax.dot_general(x, eye, (((0,), (0,)), ((), ())),
                               preferred_element_type=jnp.float32)


def _pack_body(e1t_ref, e2t_ref, et_ref, out_ref):
    out_ref[...] = jnp.concatenate(
        [_mxu_t(e1t_ref[...], HALF), _mxu_t(e2t_ref[...], HALF),
         _mxu_t(et_ref[...], HIDDEN)], axis=1)


def _pack(e1t, e2t, et, n, blk):
    nblk = (n + blk - 1) // blk
    return pl.pallas_call(
        _pack_body,
        grid=(nblk,),
        in_specs=[
            pl.BlockSpec((HALF, blk), lambda i: (0, i)),
            pl.BlockSpec((HALF, blk), lambda i: (0, i)),
            pl.BlockSpec((HIDDEN, blk), lambda i: (0, i)),
        ],
        out_specs=pl.BlockSpec((blk, 128), lambda i: (i, 0)),
        out_shape=jax.ShapeDtypeStruct((n, 128), jnp.float32),
    )(e1t, e2t, et)


def _row_block(hb, tb, rb, i, acc32, acc64):
    comp = jnp.zeros((LANES,), jnp.float32)
    dist = jnp.zeros((LANES,), jnp.float32)
    for c in range(0, HALF, LANES):
        a1 = hb[i, pl.ds(c, LANES)]
        a2 = hb[i, pl.ds(HALF + c, LANES)]
        b1 = tb[i, pl.ds(c, LANES)]
        b2 = tb[i, pl.ds(HALF + c, LANES)]
        q1 = rb[i, pl.ds(c, LANES)]
        q2 = rb[i, pl.ds(HALF + c, LANES)]
        comp = comp + (a1 * b1 + a2 * b2) * q1 + (a1 * b2 - a2 * b1) * q2
        acc32 = acc32 + a1 * a1 + a2 * a2 + b1 * b1 + b2 * b2 + q1 * q1 + q2 * q2
    for c in range(0, HIDDEN, LANES):
        x = hb[i, pl.ds(HIDDEN + c, LANES)]
        z = tb[i, pl.ds(HIDDEN + c, LANES)]
        w = rb[i, pl.ds(HIDDEN + c, LANES)]
        dist = dist + x * z * w
        acc64 = acc64 + x * x + z * z + w * w
    total = jnp.sum(comp + dist)
    return total, acc32, acc64


def _sc_body(h_hbm, t_hbm, r_hbm, ent_hbm, rel_hbm,
             res_hbm, part_hbm,
             h_v, t_v, r_v, slots, res_v, part_v, sems):
    wid = lax.axis_index("s") * NC + lax.axis_index("c")
    base = wid * ROWS_PER_W
    lane = lax.iota(jnp.int32, LANES)

    pltpu.sync_copy(h_hbm.at[pl.ds(base, ROWS_PER_W)], h_v)
    pltpu.sync_copy(t_hbm.at[pl.ds(base, ROWS_PER_W)], t_v)
    pltpu.sync_copy(r_hbm.at[pl.ds(base, ROWS_PER_W)], r_v)

    def fire(g, s):
        hb, tb, rb = slots[s]
        sl = pl.ds(g * CHUNK, CHUNK)
        return [pltpu.async_copy(ent_hbm.at[h_v.at[sl]], hb, sems[s]),
                pltpu.async_copy(ent_hbm.at[t_v.at[sl]], tb, sems[s]),
                pltpu.async_copy(rel_hbm.at[r_v.at[sl]], rb, sems[s])]

    pending = {0: fire(0, 0)}
    for g in range(NCHUNK):
        s = g % NBUF
        if g + 1 < NCHUNK:
            pending[g + 1] = fire(g + 1, (g + 1) % NBUF)
        for d in pending.pop(g):
            d.wait()
        hb, tb, rb = slots[s]

        def body(i16, carry, _hb=hb, _tb=tb, _rb=rb, _g=g):
            acc32, acc64 = carry
            res_vec = jnp.zeros((LANES,), jnp.float32)
            for k in range(LANES):
                total, acc32, acc64 = _row_block(_hb, _tb, _rb,
                                                 i16 * LANES + k,
                                                 acc32, acc64)
                res_vec = jnp.where(lane == k, total, res_vec)
            res_v[pl.ds(_g * CHUNK + i16 * LANES, LANES)] = res_vec
            return acc32, acc64

        if g == 0:
            carry = (jnp.zeros((LANES,), jnp.float32),
                     jnp.zeros((LANES,), jnp.float32))
        carry = lax.fori_loop(0, CHUNK // LANES, body, carry)

    acc32, acc64 = carry
    zero = jnp.zeros((LANES,), jnp.float32)
    part_v[pl.ds(0, LANES)] = acc32
    part_v[pl.ds(LANES, LANES)] = acc64
    for j in range(2, 8):
        part_v[pl.ds(j * LANES, LANES)] = zero
    pltpu.sync_copy(res_v, res_hbm.at[pl.ds(base, ROWS_PER_W)])
    pltpu.sync_copy(part_v, part_hbm.at[pl.ds(wid * 128, 128)])


def _make_sc_call():
    mesh = plsc.VectorSubcoreMesh(core_axis_name="c", subcore_axis_name="s",
                                  num_cores=NC)
    slot = lambda: tuple(pltpu.VMEM((CHUNK, 128), jnp.float32)
                         for _ in range(3))
    return pl.kernel(
        _sc_body,
        out_type=(jax.ShapeDtypeStruct((BATCH,), jnp.float32),
                  jax.ShapeDtypeStruct((NW * 128,), jnp.float32)),
        mesh=mesh,
        compiler_params=pltpu.CompilerParams(needs_layout_passes=False,
                                             use_tc_tiling_on_sc=True),
        scratch_types=[
            pltpu.VMEM((ROWS_PER_W,), jnp.int32),
            pltpu.VMEM((ROWS_PER_W,), jnp.int32),
            pltpu.VMEM((ROWS_PER_W,), jnp.int32),
            tuple(slot() for _ in range(NBUF)),
            pltpu.VMEM((ROWS_PER_W,), jnp.float32),
            pltpu.VMEM((128,), jnp.float32),
            tuple(pltpu.SemaphoreType.DMA for _ in range(NBUF)),
        ],
    )


def _finish_body(res_ref, y_ref, part_ref, out_ref):
    z = -y_ref[...] * res_ref[...]
    sp = jnp.maximum(z, 0.0) + jnp.log1p(jnp.exp(-jnp.abs(z)))
    loss = jnp.sum(sp) * (1.0 / BATCH)
    regul = (jnp.sum(part_ref[:, 0:LANES]) * (1.0 / (BATCH * HALF))
             + jnp.sum(part_ref[:, LANES:2 * LANES]) * (1.0 / (BATCH * HIDDEN)))
    out_ref[0, 0] = loss + LMBDA * regul


def kernel(h, t, r, y, ent1_embeddings, ent2_embeddings, ent_embeddings,
           rel1_embeddings, rel2_embeddings, rel_embeddings):
    h = h.astype(jnp.int32)
    t = t.astype(jnp.int32)
    r = r.astype(jnp.int32)
    ent_packed = _pack(ent1_embeddings.T, ent2_embeddings.T,
                       ent_embeddings.T, ENT_TOTAL, 4096)
    rel_packed = _pack(rel1_embeddings.T, rel2_embeddings.T,
                       rel_embeddings.T, REL_TOTAL, 1024)
    sc = _make_sc_call()
    res, part = sc(h, t, r, ent_packed, rel_packed)
    out = pl.pallas_call(
        _finish_body,
        out_shape=jax.ShapeDtypeStruct((1, 1), jnp.float32),
        out_specs=pl.BlockSpec(memory_space=pltpu.SMEM),
    )(res.reshape(128, 128), y.reshape(128, 128), part.reshape(NW, 128))
    return out[0, 0]


# speedup vs baseline: 1.0753x; 1.0753x over previous
import jax
import jax.numpy as jnp
from jax import lax
from jax.experimental import pallas as pl
from jax.experimental.pallas import tpu as pltpu
from jax.experimental.pallas import tpu_sc as plsc

ENT_TOTAL = 100000
REL_TOTAL = 1000
HIDDEN = 64
HALF = HIDDEN // 2
BATCH = 16384
LMBDA = 0.0001

NC = 2
NS = 16
LANES = 16
NW = NC * NS
ROWS_PER_W = BATCH // NW
CHUNK = 128
NCHUNK = ROWS_PER_W // CHUNK
NBUF = 2


def _mxu_t(x, k):
    eye = jnp.eye(k, dtype=jnp.float32)
    return jax.lax.dot_general(x, eye, (((0,), (0,)), ((), ())),
                               preferred_element_type=jnp.float32)


def _pack_body(e1t_ref, e2t_ref, et_ref, out_ref):
    out_ref[...] = jnp.concatenate(
        [_mxu_t(e1t_ref[...], HALF), _mxu_t(e2t_ref[...], HALF),
         _mxu_t(et_ref[...], HIDDEN)], axis=1)


def _pack(e1t, e2t, et, n, blk):
    nblk = (n + blk - 1) // blk
    return pl.pallas_call(
        _pack_body,
        grid=(nblk,),
        in_specs=[
            pl.BlockSpec((HALF, blk), lambda i: (0, i)),
            pl.BlockSpec((HALF, blk), lambda i: (0, i)),
            pl.BlockSpec((HIDDEN, blk), lambda i: (0, i)),
        ],
        out_specs=pl.BlockSpec((blk, 128), lambda i: (i, 0)),
        out_shape=jax.ShapeDtypeStruct((n, 128), jnp.float32),
    )(e1t, e2t, et)


def _row_block(hb, tb, rb, i, acc32, acc64):
    comp = jnp.zeros((LANES,), jnp.float32)
    dist = jnp.zeros((LANES,), jnp.float32)
    for c in range(0, HALF, LANES):
        a1 = hb[i, pl.ds(c, LANES)]
        a2 = hb[i, pl.ds(HALF + c, LANES)]
        b1 = tb[i, pl.ds(c, LANES)]
        b2 = tb[i, pl.ds(HALF + c, LANES)]
        q1 = rb[i, pl.ds(c, LANES)]
        q2 = rb[i, pl.ds(HALF + c, LANES)]
        comp = comp + (a1 * b1 + a2 * b2) * q1 + (a1 * b2 - a2 * b1) * q2
        acc32 = acc32 + a1 * a1 + a2 * a2 + b1 * b1 + b2 * b2 + q1 * q1 + q2 * q2
    for c in range(0, HIDDEN, LANES):
        x = hb[i, pl.ds(HIDDEN + c, LANES)]
        z = tb[i, pl.ds(HIDDEN + c, LANES)]
        w = rb[i, pl.ds(HIDDEN + c, LANES)]
        dist = dist + x * z * w
        acc64 = acc64 + x * x + z * z + w * w
    total = jnp.sum(comp + dist)
    return total, acc32, acc64


def _sc_body(h_hbm, t_hbm, r_hbm, ent_hbm, rel_hbm,
             res_hbm, part_hbm,
             h_v, t_v, r_v, slots, res_v, part_v, sems):
    wid = lax.axis_index("s") * NC + lax.axis_index("c")
    base = wid * ROWS_PER_W
    lane = lax.iota(jnp.int32, LANES)

    pltpu.sync_copy(h_hbm.at[pl.ds(base, ROWS_PER_W)], h_v)
    pltpu.sync_copy(t_hbm.at[pl.ds(base, ROWS_PER_W)], t_v)
    pltpu.sync_copy(r_hbm.at[pl.ds(base, ROWS_PER_W)], r_v)

    def fire(g, s):
        hb, tb, rb = slots[s]
        sl = pl.ds(g * CHUNK, CHUNK)
        return [pltpu.async_copy(ent_hbm.at[h_v.at[sl]], hb, sems[s]),
                pltpu.async_copy(ent_hbm.at[t_v.at[sl]], tb, sems[s]),
                pltpu.async_copy(rel_hbm.at[r_v.at[sl]], rb, sems[s])]

    pending = {0: fire(0, 0)}
    for g in range(NCHUNK):
        s = g % NBUF
        if g + 1 < NCHUNK:
            pending[g + 1] = fire(g + 1, (g + 1) % NBUF)
        for d in pending.pop(g):
            d.wait()
        hb, tb, rb = slots[s]

        def body(i16, carry, _hb=hb, _tb=tb, _rb=rb, _g=g):
            acc32, acc64 = carry
            res_vec = jnp.zeros((LANES,), jnp.float32)
            for k in range(LANES):
                total, acc32, acc64 = _row_block(_hb, _tb, _rb,
                                                 i16 * LANES + k,
                                                 acc32, acc64)
                res_vec = jnp.where(lane == k, total, res_vec)
            res_v[pl.ds(_g * CHUNK + i16 * LANES, LANES)] = res_vec
            return acc32, acc64

        if g == 0:
            carry = (jnp.zeros((LANES,), jnp.float32),
                     jnp.zeros((LANES,), jnp.float32))
        carry = lax.fori_loop(0, CHUNK // LANES, body, carry)

    acc32, acc64 = carry
    zero = jnp.zeros((LANES,), jnp.float32)
    part_v[pl.ds(0, LANES)] = acc32
    part_v[pl.ds(LANES, LANES)] = acc64
    for j in range(2, 8):
        part_v[pl.ds(j * LANES, LANES)] = zero
    pltpu.sync_copy(res_v, res_hbm.at[pl.ds(base, ROWS_PER_W)])
    pltpu.sync_copy(part_v, part_hbm.at[pl.ds(wid * 128, 128)])


def _make_sc_call():
    mesh = plsc.VectorSubcoreMesh(core_axis_name="c", subcore_axis_name="s")
    slot = lambda: tuple(pltpu.VMEM((CHUNK, 128), jnp.float32)
                         for _ in range(3))
    return pl.kernel(
        _sc_body,
        out_type=(jax.ShapeDtypeStruct((BATCH,), jnp.float32),
                  jax.ShapeDtypeStruct((NW * 128,), jnp.float32)),
        mesh=mesh,
        compiler_params=pltpu.CompilerParams(needs_layout_passes=False,
                                             use_tc_tiling_on_sc=True),
        scratch_types=[
            pltpu.VMEM((ROWS_PER_W,), jnp.int32),
            pltpu.VMEM((ROWS_PER_W,), jnp.int32),
            pltpu.VMEM((ROWS_PER_W,), jnp.int32),
            tuple(slot() for _ in range(NBUF)),
            pltpu.VMEM((ROWS_PER_W,), jnp.float32),
            pltpu.VMEM((128,), jnp.float32),
            tuple(pltpu.SemaphoreType.DMA for _ in range(NBUF)),
        ],
    )


def _finish_body(res_ref, y_ref, part_ref, out_ref):
    z = -y_ref[...] * res_ref[...]
    sp = jnp.maximum(z, 0.0) + jnp.log1p(jnp.exp(-jnp.abs(z)))
    loss = jnp.sum(sp) * (1.0 / BATCH)
    regul = (jnp.sum(part_ref[:, 0:LANES]) * (1.0 / (BATCH * HALF))
             + jnp.sum(part_ref[:, LANES:2 * LANES]) * (1.0 / (BATCH * HIDDEN)))
    out_ref[0, 0] = loss + LMBDA * regul


def kernel(h, t, r, y, ent1_embeddings, ent2_embeddings, ent_embeddings,
           rel1_embeddings, rel2_embeddings, rel_embeddings):
    h = h.astype(jnp.int32)
    t = t.astype(jnp.int32)
    r = r.astype(jnp.int32)
    ent_packed = _pack(ent1_embeddings.T, ent2_embeddings.T,
                       ent_embeddings.T, ENT_TOTAL, 4096)
    rel_packed = _pack(rel1_embeddings.T, rel2_embeddings.T,
                       rel_embeddings.T, REL_TOTAL, 1024)
    sc = _make_sc_call()
    res, part = sc(h, t, r, ent_packed, rel_packed)
    out = pl.pallas_call(
        _finish_body,
        out_shape=jax.ShapeDtypeStruct((1, 1), jnp.float32),
        out_specs=pl.BlockSpec(memory_space=pltpu.SMEM),
    )(res.reshape(128, 128), y.reshape(128, 128), part.reshape(NW, 128))
    return out[0, 0]


# speedup vs baseline: 1.0793x; 1.0038x over previous
import jax
import jax.numpy as jnp
from jax import lax
from jax.experimental import pallas as pl
from jax.experimental.pallas import tpu as pltpu
from jax.experimental.pallas import tpu_sc as plsc

ENT_TOTAL = 100000
REL_TOTAL = 1000
HIDDEN = 64
HALF = HIDDEN // 2
BATCH = 16384
LMBDA = 0.0001

NC = 2
NS = 16
LANES = 16
NW = NC * NS
ROWS_PER_W = BATCH // NW
CHUNK = 128
NCHUNK = ROWS_PER_W // CHUNK
NBUF = 2


def _mxu_t(x, k):
    eye = jnp.eye(k, dtype=jnp.float32)
    return jax.lax.dot_general(x, eye, (((0,), (0,)), ((), ())),
                               preferred_element_type=jnp.float32)


def _pack_body(e1t_ref, e2t_ref, et_ref, out_ref):
    out_ref[...] = jnp.concatenate(
        [e1t_ref[...].T, e2t_ref[...].T, et_ref[...].T], axis=1)


def _pack(e1t, e2t, et, n, blk):
    nblk = (n + blk - 1) // blk
    return pl.pallas_call(
        _pack_body,
        grid=(nblk,),
        in_specs=[
            pl.BlockSpec((HALF, blk), lambda i: (0, i)),
            pl.BlockSpec((HALF, blk), lambda i: (0, i)),
            pl.BlockSpec((HIDDEN, blk), lambda i: (0, i)),
        ],
        out_specs=pl.BlockSpec((blk, 128), lambda i: (i, 0)),
        out_shape=jax.ShapeDtypeStruct((n, 128), jnp.float32),
    )(e1t, e2t, et)


def _row_block(hb, tb, rb, i, acc32, acc64):
    comp = jnp.zeros((LANES,), jnp.float32)
    dist = jnp.zeros((LANES,), jnp.float32)
    for c in range(0, HALF, LANES):
        a1 = hb[i, pl.ds(c, LANES)]
        a2 = hb[i, pl.ds(HALF + c, LANES)]
        b1 = tb[i, pl.ds(c, LANES)]
        b2 = tb[i, pl.ds(HALF + c, LANES)]
        q1 = rb[i, pl.ds(c, LANES)]
        q2 = rb[i, pl.ds(HALF + c, LANES)]
        comp = comp + (a1 * b1 + a2 * b2) * q1 + (a1 * b2 - a2 * b1) * q2
        acc32 = acc32 + a1 * a1 + a2 * a2 + b1 * b1 + b2 * b2 + q1 * q1 + q2 * q2
    for c in range(0, HIDDEN, LANES):
        x = hb[i, pl.ds(HIDDEN + c, LANES)]
        z = tb[i, pl.ds(HIDDEN + c, LANES)]
        w = rb[i, pl.ds(HIDDEN + c, LANES)]
        dist = dist + x * z * w
        acc64 = acc64 + x * x + z * z + w * w
    total = jnp.sum(comp + dist)
    return total, acc32, acc64


def _sc_body(h_hbm, t_hbm, r_hbm, ent_hbm, rel_hbm,
             res_hbm, part_hbm,
             h_v, t_v, r_v, slots, res_v, part_v, sems):
    wid = lax.axis_index("s") * NC + lax.axis_index("c")
    base = wid * ROWS_PER_W
    lane = lax.iota(jnp.int32, LANES)

    pltpu.sync_copy(h_hbm.at[pl.ds(base, ROWS_PER_W)], h_v)
    pltpu.sync_copy(t_hbm.at[pl.ds(base, ROWS_PER_W)], t_v)
    pltpu.sync_copy(r_hbm.at[pl.ds(base, ROWS_PER_W)], r_v)

    def fire(g, s):
        hb, tb, rb = slots[s]
        sl = pl.ds(g * CHUNK, CHUNK)
        return [pltpu.async_copy(ent_hbm.at[h_v.at[sl]], hb, sems[s]),
                pltpu.async_copy(ent_hbm.at[t_v.at[sl]], tb, sems[s]),
                pltpu.async_copy(rel_hbm.at[r_v.at[sl]], rb, sems[s])]

    pending = {0: fire(0, 0)}
    for g in range(NCHUNK):
        s = g % NBUF
        if g + 1 < NCHUNK:
            pending[g + 1] = fire(g + 1, (g + 1) % NBUF)
        for d in pending.pop(g):
            d.wait()
        hb, tb, rb = slots[s]

        def body(i16, carry, _hb=hb, _tb=tb, _rb=rb, _g=g):
            acc32, acc64 = carry
            res_vec = jnp.zeros((LANES,), jnp.float32)
            for k in range(LANES):
                total, acc32, acc64 = _row_block(_hb, _tb, _rb,
                                                 i16 * LANES + k,
                                                 acc32, acc64)
                res_vec = jnp.where(lane == k, total, res_vec)
            res_v[pl.ds(_g * CHUNK + i16 * LANES, LANES)] = res_vec
            return acc32, acc64

        if g == 0:
            carry = (jnp.zeros((LANES,), jnp.float32),
                     jnp.zeros((LANES,), jnp.float32))
        carry = lax.fori_loop(0, CHUNK // LANES, body, carry)

    acc32, acc64 = carry
    zero = jnp.zeros((LANES,), jnp.float32)
    part_v[pl.ds(0, LANES)] = acc32
    part_v[pl.ds(LANES, LANES)] = acc64
    for j in range(2, 8):
        part_v[pl.ds(j * LANES, LANES)] = zero
    pltpu.sync_copy(res_v, res_hbm.at[pl.ds(base, ROWS_PER_W)])
    pltpu.sync_copy(part_v, part_hbm.at[pl.ds(wid * 128, 128)])


def _make_sc_call():
    mesh = plsc.VectorSubcoreMesh(core_axis_name="c", subcore_axis_name="s")
    slot = lambda: tuple(pltpu.VMEM((CHUNK, 128), jnp.float32)
                         for _ in range(3))
    return pl.kernel(
        _sc_body,
        out_type=(jax.ShapeDtypeStruct((BATCH,), jnp.float32),
                  jax.ShapeDtypeStruct((NW * 128,), jnp.float32)),
        mesh=mesh,
        compiler_params=pltpu.CompilerParams(needs_layout_passes=False,
                                             use_tc_tiling_on_sc=True),
        scratch_types=[
            pltpu.VMEM((ROWS_PER_W,), jnp.int32),
            pltpu.VMEM((ROWS_PER_W,), jnp.int32),
            pltpu.VMEM((ROWS_PER_W,), jnp.int32),
            tuple(slot() for _ in range(NBUF)),
            pltpu.VMEM((ROWS_PER_W,), jnp.float32),
            pltpu.VMEM((128,), jnp.float32),
            tuple(pltpu.SemaphoreType.DMA for _ in range(NBUF)),
        ],
    )


def _finish_body(res_ref, y_ref, part_ref, out_ref):
    z = -y_ref[...] * res_ref[...]
    sp = jnp.maximum(z, 0.0) + jnp.log1p(jnp.exp(-jnp.abs(z)))
    loss = jnp.sum(sp) * (1.0 / BATCH)
    regul = (jnp.sum(part_ref[:, 0:LANES]) * (1.0 / (BATCH * HALF))
             + jnp.sum(part_ref[:, LANES:2 * LANES]) * (1.0 / (BATCH * HIDDEN)))
    out_ref[0, 0] = loss + LMBDA * regul


def kernel(h, t, r, y, ent1_embeddings, ent2_embeddings, ent_embeddings,
           rel1_embeddings, rel2_embeddings, rel_embeddings):
    h = h.astype(jnp.int32)
    t = t.astype(jnp.int32)
    r = r.astype(jnp.int32)
    ent_packed = _pack(ent1_embeddings.T, ent2_embeddings.T,
                       ent_embeddings.T, ENT_TOTAL, 4096)
    rel_packed = _pack(rel1_embeddings.T, rel2_embeddings.T,
                       rel_embeddings.T, REL_TOTAL, 1024)
    sc = _make_sc_call()
    res, part = sc(h, t, r, ent_packed, rel_packed)
    out = pl.pallas_call(
        _finish_body,
        out_shape=jax.ShapeDtypeStruct((1, 1), jnp.float32),
        out_specs=pl.BlockSpec(memory_space=pltpu.SMEM),
    )(res.reshape(128, 128), y.reshape(128, 128), part.reshape(NW, 128))
    return out[0, 0]
